# baseline (speedup 1.0000x reference)
import functools

import jax
import jax.numpy as jnp
from jax import lax
from jax.experimental import pallas as pl
from jax.experimental.pallas import tpu as pltpu
from jax.experimental.pallas import tpu_sc as plsc

_L = 16


def _pos_sc(rowT, colT, B):
    d, h = rowT.shape
    w = colT.shape[1]
    hw = h * w
    info = plsc.get_sparse_core_info()
    NC, NS = info.num_cores, info.num_subcores
    NW = NC * NS
    assert d % NW == 0 and w % _L == 0
    d_per = d // NW

    mesh = plsc.VectorSubcoreMesh(core_axis_name="c", subcore_axis_name="s")

    @functools.partial(
        pl.kernel,
        out_type=jax.ShapeDtypeStruct((B, d, hw), jnp.float32),
        mesh=mesh,
        scratch_types=[
            pltpu.VMEM((d_per, h), jnp.float32),
            pltpu.VMEM((d_per, w), jnp.float32),
            pltpu.VMEM((d_per, hw), jnp.float32),
            pltpu.SemaphoreType.DMA,
        ],
    )
    def k(rowT_hbm, colT_hbm, out_hbm, rowv, colv, posv, sem):
        wid = lax.axis_index("s") * NC + lax.axis_index("c")
        base = wid * d_per
        pltpu.sync_copy(rowT_hbm.at[pl.ds(base, d_per)], rowv)
        pltpu.sync_copy(colT_hbm.at[pl.ds(base, d_per)], colv)
        for dd in range(d_per):
            cvecs = [colv[dd, pl.ds(jv * _L, _L)] for jv in range(w // _L)]
            rvecs = [rowv[dd, pl.ds(iv * _L, _L)] for iv in range(h // _L)]
            for i in range(h):
                idx = jnp.full((_L, 1), i % _L, jnp.int32)
                rbc = lax.gather(
                    rvecs[i // _L], idx,
                    lax.GatherDimensionNumbers(
                        offset_dims=(), collapsed_slice_dims=(0,),
                        start_index_map=(0,)),
                    slice_sizes=(1,),
                    mode=lax.GatherScatterMode.PROMISE_IN_BOUNDS)
                for jv, cv in enumerate(cvecs):
                    posv[dd, pl.ds(i * w + jv * _L, _L)] = rbc + cv
        handles = [
            pltpu.async_copy(posv, out_hbm.at[b, pl.ds(base, d_per), :], sem)
            for b in range(B)
        ]
        for hnd in handles:
            hnd.wait()

    return k(rowT, colT)


def kernel(x, mask, row_embed, col_embed):
    B = x.shape[0]
    h, w = x.shape[-2], x.shape[-1]
    d = row_embed.shape[-1]
    rowT = row_embed.T
    colT = col_embed.T
    out = _pos_sc(rowT, colT, B)
    return out.reshape(B, d, h, w)


# speedup vs baseline: 1.4748x; 1.4748x over previous
import jax
import jax.numpy as jnp
from jax.experimental import pallas as pl
from jax.experimental.pallas import tpu as pltpu

_NSRC = 4


def _body(row_ref, col_ref, o_ref, s_ref, sems):
    d, h = row_ref.shape
    w = col_ref.shape[1]
    hw = h * w
    B = o_ref.shape[0]

    p_i = jax.lax.broadcasted_iota(jnp.int32, (h, hw), 1) // w
    p_j = jax.lax.broadcasted_iota(jnp.int32, (w, hw), 1) % w
    ii = jax.lax.broadcasted_iota(jnp.int32, (h, hw), 0)
    jj = jax.lax.broadcasted_iota(jnp.int32, (w, hw), 0)
    R = (p_i == ii).astype(jnp.float32)
    C = (p_j == jj).astype(jnp.float32)
    s_ref[0] = (
        jnp.dot(row_ref[...], R, preferred_element_type=jnp.float32,
                precision=jax.lax.Precision.HIGHEST)
        + jnp.dot(col_ref[...], C, preferred_element_type=jnp.float32,
                  precision=jax.lax.Precision.HIGHEST)
    )
    reps = [pltpu.make_async_copy(s_ref.at[0], s_ref.at[k], sems.at[k])
            for k in range(1, _NSRC)]
    for r in reps:
        r.start()
    for r in reps:
        r.wait()

    copies = [
        pltpu.make_async_copy(s_ref.at[b % _NSRC], o_ref.at[b], sems.at[b])
        for b in range(B)
    ]
    for c in copies:
        c.start()
    for c in copies:
        c.wait()


def kernel(x, mask, row_embed, col_embed):
    B = x.shape[0]
    h, w = x.shape[-2], x.shape[-1]
    d = row_embed.shape[-1]
    rowT = row_embed.T
    colT = col_embed.T
    out = pl.pallas_call(
        _body,
        in_specs=[
            pl.BlockSpec((d, h), lambda: (0, 0)),
            pl.BlockSpec((d, w), lambda: (0, 0)),
        ],
        out_specs=pl.BlockSpec(memory_space=pl.ANY),
        out_shape=jax.ShapeDtypeStruct((B, d, h * w), jnp.float32),
        scratch_shapes=[
            pltpu.VMEM((_NSRC, d, h * w), jnp.float32),
            pltpu.SemaphoreType.DMA((max(B, _NSRC),)),
        ],
    )(rowT, colT)
    return out.reshape(B, d, h, w)
